# baseline (speedup 1.0000x reference)
import functools

import jax
import jax.numpy as jnp
from jax import lax
from jax.experimental import pallas as pl
from jax.experimental.pallas import tpu as pltpu
from jax.experimental.pallas import tpu_sc as plsc

B = 16
T = 1024
D = 128
E = 1024

EB = 128


def _codebook_body(emb_ref, smul_ref, sadd_ref, cb_ref):
    emb = emb_ref[...]
    raw = lax.dot_general(
        emb.reshape(EB * D, D), smul_ref[...],
        (((1,), (0,)), ((), ())), preferred_element_type=jnp.float32)
    raw = raw.reshape(EB, D, B)
    norm = jnp.sqrt(jnp.sum(raw * raw, axis=1, keepdims=True))
    cb_ref[...] = raw / norm + sadd_ref[...][None, :, :]


def _build_codebook(embedding, smul, sadd):
    return pl.pallas_call(
        _codebook_body,
        grid=(E // EB,),
        in_specs=[
            pl.BlockSpec((EB, D, D), lambda i: (i, 0, 0)),
            pl.BlockSpec((D, B), lambda i: (0, 0)),
            pl.BlockSpec((D, B), lambda i: (0, 0)),
        ],
        out_specs=pl.BlockSpec((EB, D, B), lambda i: (i, 0, 0)),
        out_shape=jax.ShapeDtypeStruct((E, D, B), jnp.float32),
    )(embedding, smul, sadd)


def _argmin_body(x_ref, cb_ref, ind_ref, indflat_ref):
    x = x_ref[0]
    cb = cb_ref[0]
    scores = lax.dot_general(
        x, cb, (((1,), (1,)), ((), ())), preferred_element_type=jnp.float32)
    xsq = jnp.sum(x * x, axis=1, keepdims=True)
    esq = jnp.sum(cb * cb, axis=1)
    neg = -(xsq - 2.0 * scores + esq[None, :])
    m = jnp.max(neg, axis=1, keepdims=True)
    iota = lax.broadcasted_iota(jnp.int32, (T, E), 1)
    ind = jnp.min(jnp.where(neg == m, iota, E), axis=1)
    ind_ref[0, 0, :] = ind
    indflat_ref[0, 0, :] = ind + pl.program_id(0) * E


def _distance_argmin(inp, embed2):
    return pl.pallas_call(
        _argmin_body,
        grid=(B,),
        in_specs=[
            pl.BlockSpec((1, T, D), lambda b: (b, 0, 0)),
            pl.BlockSpec((1, E, D), lambda b: (b, 0, 0)),
        ],
        out_specs=[
            pl.BlockSpec((1, 1, T), lambda b: (b, 0, 0)),
            pl.BlockSpec((1, 1, T), lambda b: (b, 0, 0)),
        ],
        out_shape=[
            jax.ShapeDtypeStruct((B, 1, T), jnp.int32),
            jax.ShapeDtypeStruct((B, 1, T), jnp.int32),
        ],
    )(inp, embed2)


NC = 2
NS = 16
NW = NC * NS
ROWS = B * T
RPW = ROWS // NW
CH = 128
NCH = RPW // CH


def _gather_body(table_hbm, idx_hbm, out_hbm, idx_v, rows_v, sem):
    wid = lax.axis_index("s") * NC + lax.axis_index("c")
    pltpu.sync_copy(idx_hbm.at[pl.ds(wid * NCH, NCH)], idx_v)
    for j in range(NCH):
        pltpu.async_copy(
            table_hbm.at[idx_v.at[j]],
            rows_v.at[pl.ds(j * CH, CH)], sem).wait()
    pltpu.sync_copy(rows_v, out_hbm.at[pl.ds(wid * RPW, RPW)])


@functools.cache
def _sc_gather_fn():
    return pl.kernel(
        _gather_body,
        out_type=jax.ShapeDtypeStruct((ROWS, D), jnp.float32),
        mesh=plsc.VectorSubcoreMesh(core_axis_name="c", subcore_axis_name="s",
                                    num_cores=NC, num_subcores=NS),
        scratch_types=[
            pltpu.VMEM((NCH, CH), jnp.int32),
            pltpu.VMEM((RPW, D), jnp.float32),
            pltpu.SemaphoreType.DMA,
        ],
    )


def _combine_body(q_ref, x_ref, out_ref, acc_ref):
    q = q_ref[...]
    x = x_ref[...]
    r = q - x
    out_ref[...] = (q + (x + r)) * 0.5
    s = jnp.sum(r * r)

    @pl.when(pl.program_id(0) == 0)
    def _():
        acc_ref[...] = jnp.zeros_like(acc_ref[...])

    acc_ref[...] += jnp.full((8, 128), s, jnp.float32)


def _combine(quantize, inp):
    return pl.pallas_call(
        _combine_body,
        grid=(B,),
        in_specs=[
            pl.BlockSpec((1, T, D), lambda b: (b, 0, 0)),
            pl.BlockSpec((1, T, D), lambda b: (b, 0, 0)),
        ],
        out_specs=[
            pl.BlockSpec((1, T, D), lambda b: (b, 0, 0)),
            pl.BlockSpec((8, 128), lambda b: (0, 0)),
        ],
        out_shape=[
            jax.ShapeDtypeStruct((B, T, D), jnp.float32),
            jax.ShapeDtypeStruct((8, 128), jnp.float32),
        ],
    )(quantize, inp)


def kernel(input, speaker_embedding, embedding):
    smul = speaker_embedding[2].T
    sadd = speaker_embedding[1].T
    codebook = _build_codebook(embedding, smul, sadd)
    embed2 = jnp.transpose(codebook, (2, 0, 1))
    ind3, indflat3 = _distance_argmin(input, embed2)
    embed_ind = ind3.reshape(B, T)
    table = embed2.reshape(B * E, D)
    idx2d = indflat3.reshape(NW * NCH, CH)
    quantize = _sc_gather_fn()(table, idx2d).reshape(B, T, D)
    out0, acc = _combine(quantize, input)
    diff = acc[0, 0] / jnp.float32(B * T * D)
    return (out0, diff, embed_ind)


# speedup vs baseline: 1.4541x; 1.4541x over previous
import functools

import jax
import jax.numpy as jnp
from jax import lax
from jax.experimental import pallas as pl
from jax.experimental.pallas import tpu as pltpu
from jax.experimental.pallas import tpu_sc as plsc

B = 16
T = 1024
D = 128
E = 1024

EB = 128


def _codebook_body(emb_ref, smul_ref, sadd_ref, cb_ref, esq_ref):
    emb = emb_ref[...]
    raw = lax.dot_general(
        emb.reshape(EB * D, D), smul_ref[...],
        (((1,), (0,)), ((), ())), preferred_element_type=jnp.float32)
    raw = raw.reshape(EB, D, B)
    norm = jnp.sqrt(jnp.sum(raw * raw, axis=1, keepdims=True))
    cb = raw / norm + sadd_ref[...][None, :, :]
    tr = jnp.transpose(cb, (2, 0, 1))
    cb_ref[...] = tr
    esq = jnp.sum(tr * tr, axis=2, keepdims=True)
    esq_ref[...] = esq.reshape(B, 1, EB)


def _build_codebook(embedding, smul, sadd):
    return pl.pallas_call(
        _codebook_body,
        grid=(E // EB,),
        in_specs=[
            pl.BlockSpec((EB, D, D), lambda i: (i, 0, 0)),
            pl.BlockSpec((D, B), lambda i: (0, 0)),
            pl.BlockSpec((D, B), lambda i: (0, 0)),
        ],
        out_specs=[
            pl.BlockSpec((B, EB, D), lambda i: (0, i, 0)),
            pl.BlockSpec((B, 1, EB), lambda i: (0, 0, i)),
        ],
        out_shape=[
            jax.ShapeDtypeStruct((B, E, D), jnp.float32),
            jax.ShapeDtypeStruct((B, 1, E), jnp.float32),
        ],
    )(embedding, smul, sadd)


def _argmin_body(x_ref, cb_ref, esq_ref, ind_ref, indflat_ref):
    x = x_ref[0]
    cb = cb_ref[0]
    xm2 = x * jnp.float32(-2.0)
    s2 = lax.dot_general(
        xm2, cb, (((1,), (1,)), ((), ())), preferred_element_type=jnp.float32)
    xsq = jnp.sum(x * x, axis=1, keepdims=True)
    dist = (xsq + s2) + esq_ref[0]
    m = jnp.min(dist, axis=1, keepdims=True)
    iota = lax.broadcasted_iota(jnp.int32, (T, E), 1)
    ind = jnp.min(jnp.where(dist == m, iota, E), axis=1)
    ind_ref[0, 0, :] = ind
    indflat_ref[0, 0, :] = ind + pl.program_id(0) * E


def _distance_argmin(inp, embed2, esq):
    return pl.pallas_call(
        _argmin_body,
        grid=(B,),
        in_specs=[
            pl.BlockSpec((1, T, D), lambda b: (b, 0, 0)),
            pl.BlockSpec((1, E, D), lambda b: (b, 0, 0)),
            pl.BlockSpec((1, 1, E), lambda b: (b, 0, 0)),
        ],
        out_specs=[
            pl.BlockSpec((1, 1, T), lambda b: (b, 0, 0)),
            pl.BlockSpec((1, 1, T), lambda b: (b, 0, 0)),
        ],
        out_shape=[
            jax.ShapeDtypeStruct((B, 1, T), jnp.int32),
            jax.ShapeDtypeStruct((B, 1, T), jnp.int32),
        ],
    )(inp, embed2, esq)


NC = 2
NS = 16
NW = NC * NS
ROWS = B * T
RPW = ROWS // NW
CH = 128
NCH = RPW // CH
VPR = D // 16


def _gather_body(table_hbm, idx_hbm, x_hbm, out_hbm, part_hbm,
                 idx_v, rows_v, x_v, acc_v, sem, xsem):
    wid = lax.axis_index("s") * NC + lax.axis_index("c")
    base = wid * RPW
    pltpu.sync_copy(idx_hbm.at[pl.ds(wid * NCH, NCH)], idx_v)

    def fire(j):
        slot = j % 2
        g = pltpu.async_copy(table_hbm.at[idx_v.at[j]], rows_v.at[slot], sem)
        x = pltpu.async_copy(
            x_hbm.at[pl.ds(base + j * CH, CH)], x_v.at[slot], xsem)
        return g, x

    inflight = [fire(0), fire(1)]
    acc_v[...] = jnp.zeros((16,), jnp.float32)

    def chunk_work(slot):
        def row_work(r, _):
            def vec_work(k, acc):
                q = rows_v[slot, r, pl.ds(k * 16, 16)]
                xv = x_v[slot, r, pl.ds(k * 16, 16)]
                rr = q - xv
                rows_v[slot, r, pl.ds(k * 16, 16)] = (q + (xv + rr)) * 0.5
                return acc + rr * rr
            acc = lax.fori_loop(0, VPR, vec_work, jnp.zeros((16,), jnp.float32))
            acc_v[...] += acc
            return _
        lax.fori_loop(0, CH, row_work, 0)

    for j in range(NCH):
        slot = j % 2
        g, x = inflight[slot]
        g.wait()
        x.wait()
        chunk_work(slot)
        pltpu.sync_copy(rows_v.at[slot], out_hbm.at[pl.ds(base + j * CH, CH)])
        if j + 2 < NCH:
            inflight[slot] = fire(j + 2)
    pltpu.sync_copy(acc_v, part_hbm.at[wid])


@functools.cache
def _sc_gather_fn():
    return pl.kernel(
        _gather_body,
        out_type=[
            jax.ShapeDtypeStruct((ROWS, D), jnp.float32),
            jax.ShapeDtypeStruct((NW, 16), jnp.float32),
        ],
        mesh=plsc.VectorSubcoreMesh(core_axis_name="c", subcore_axis_name="s",
                                    num_cores=NC, num_subcores=NS),
        scratch_types=[
            pltpu.VMEM((NCH, CH), jnp.int32),
            pltpu.VMEM((2, CH, D), jnp.float32),
            pltpu.VMEM((2, CH, D), jnp.float32),
            pltpu.VMEM((16,), jnp.float32),
            pltpu.SemaphoreType.DMA,
            pltpu.SemaphoreType.DMA,
        ],
    )


def kernel(input, speaker_embedding, embedding):
    smul = speaker_embedding[2].T
    sadd = speaker_embedding[1].T
    embed2, esq = _build_codebook(embedding, smul, sadd)
    ind3, indflat3 = _distance_argmin(input, embed2, esq)
    embed_ind = ind3.reshape(B, T)
    table = embed2.reshape(B * E, D)
    idx2d = indflat3.reshape(NW * NCH, CH)
    x_flat = input.reshape(ROWS, D)
    out_flat, parts = _sc_gather_fn()(table, idx2d, x_flat)
    out0 = out_flat.reshape(B, T, D)
    diff = jnp.sum(parts) / jnp.float32(ROWS * D)
    return (out0, diff, embed_ind)
